# Initial kernel scaffold; baseline (speedup 1.0000x reference)
import jax
import jax.numpy as jnp
from jax.experimental import pallas as pl


def kernel(contours, scores, boxes, uncertainties):
    raise NotImplementedError("write your pallas kernel here")


# speedup vs baseline: 71.7641x; 71.7641x over previous
import jax
import jax.numpy as jnp
from jax.experimental import pallas as pl
from jax.experimental.pallas import tpu as pltpu

_N = 5000
_IOU_T = 0.5
_B = 512
_NPAD = 5120
_NB = _NPAD // _B


def _weight_kernel(scores_ref, unc_ref, w_ref):
    m = jnp.mean(unc_ref[...], axis=0, keepdims=True)
    w_ref[...] = scores_ref[...] * (1.0 - jax.nn.sigmoid(m))


def _nms_kernel(boxes_ref, bt_ref, keep_ref, m_ref, supp_ref):
    cx0 = bt_ref[0:1, :]
    cy0 = bt_ref[1:2, :]
    cx1 = bt_ref[2:3, :]
    cy1 = bt_ref[3:4, :]
    carea = (cx1 - cx0) * (cy1 - cy0)
    col_idx = jax.lax.broadcasted_iota(jnp.int32, (1, _NPAD), 1)
    supp_ref[...] = jnp.zeros((1, _NPAD), jnp.float32)

    def block_body(bi, carry):
        r0 = bi * _B
        rows = boxes_ref[pl.ds(r0, _B), :]
        rx0 = rows[:, 0:1]
        ry0 = rows[:, 1:2]
        rx1 = rows[:, 2:3]
        ry1 = rows[:, 3:4]
        rarea = (rx1 - rx0) * (ry1 - ry0)
        w = jnp.maximum(jnp.minimum(rx1, cx1) - jnp.maximum(rx0, cx0), 0.0)
        h = jnp.maximum(jnp.minimum(ry1, cy1) - jnp.maximum(ry0, cy0), 0.0)
        inter = w * h
        union = rarea + carea - inter
        row_idx = r0 + jax.lax.broadcasted_iota(jnp.int32, (_B, 1), 0)
        m_ref[...] = jnp.where(
            (inter > _IOU_T * union) & (col_idx > row_idx), 1.0, 0.0
        )
        active = 1.0 - supp_ref[:, pl.ds(r0, _B)]

        def w_cond(c):
            return c[1]

        def w_body(c):
            k, _ = c
            t = jnp.dot(
                k, m_ref[:, pl.ds(r0, _B)], preferred_element_type=jnp.float32
            )
            k_new = active * jnp.where(t < 0.5, 1.0, 0.0)
            return k_new, jnp.any(k_new != k)

        k, _ = jax.lax.while_loop(w_cond, w_body, (active, jnp.bool_(True)))
        supp_ref[:, pl.ds(r0, _B)] = 1.0 - k
        tall = jnp.dot(k, m_ref[...], preferred_element_type=jnp.float32)
        supp_ref[...] = jnp.maximum(
            supp_ref[...],
            jnp.where((tall > 0.5) & (col_idx >= r0 + _B), 1.0, 0.0),
        )
        return carry

    jax.lax.fori_loop(0, _NB, block_body, 0)
    keep_ref[...] = 1.0 - supp_ref[...]


def _mask_kernel(raw_ref, keep_ref, out_ref):
    out_ref[...] = raw_ref[...] * keep_ref[...]


def kernel(contours, scores, boxes, uncertainties):
    scores_row = scores.reshape(1, _N)
    unc_t = uncertainties.T
    w_row = pl.pallas_call(
        _weight_kernel,
        out_shape=jax.ShapeDtypeStruct((1, _N), jnp.float32),
    )(scores_row, unc_t)
    wflat = w_row.reshape(_N)
    order = jnp.argsort(-wflat)
    boxes_sorted = boxes[order]
    boxes_s = jnp.zeros((_NPAD, 4), jnp.float32).at[:_N].set(boxes_sorted)
    boxes_t = jnp.zeros((8, _NPAD), jnp.float32).at[:4, :_N].set(boxes_sorted.T)
    keep_row = pl.pallas_call(
        _nms_kernel,
        out_shape=jax.ShapeDtypeStruct((1, _NPAD), jnp.float32),
        scratch_shapes=[
            pltpu.VMEM((_B, _NPAD), jnp.float32),
            pltpu.VMEM((1, _NPAD), jnp.float32),
        ],
    )(boxes_s, boxes_t)
    keep = jnp.zeros(_N, jnp.float32).at[order].set(keep_row[0, :_N])
    raw = jnp.concatenate(
        [
            boxes,
            scores[:, None],
            uncertainties,
            wflat[:, None],
            contours.reshape(_N, -1),
        ],
        axis=1,
    )
    out = pl.pallas_call(
        _mask_kernel,
        out_shape=jax.ShapeDtypeStruct((_N, 74), jnp.float32),
    )(raw, keep[:, None])
    return out


# speedup vs baseline: 84.3493x; 1.1754x over previous
import jax
import jax.numpy as jnp
from jax.experimental import pallas as pl
from jax.experimental.pallas import tpu as pltpu

_N = 5000
_B = 512
_NPAD = 5120
_NB = _NPAD // _B


def _weight_kernel(scores_ref, unc_ref, w_ref):
    m = jnp.mean(unc_ref[...], axis=0, keepdims=True)
    w_ref[...] = scores_ref[...] * (1.0 - jax.nn.sigmoid(m))


def _nms_kernel(boxes_ref, bt_ref, keep_ref, tile_ref, supp_ref):
    tri = (
        jax.lax.broadcasted_iota(jnp.int32, (_B, _B), 1)
        > jax.lax.broadcasted_iota(jnp.int32, (_B, _B), 0)
    )
    supp_ref[...] = jnp.zeros((1, _NPAD), jnp.float32)

    def iou_bin(r0, c0):
        rows = boxes_ref[pl.ds(r0, _B), :]
        rx0 = rows[:, 0:1]
        ry0 = rows[:, 1:2]
        rx1 = rows[:, 2:3]
        ry1 = rows[:, 3:4]
        rarea = (rx1 - rx0) * (ry1 - ry0)
        cx0 = bt_ref[0:1, pl.ds(c0, _B)]
        cy0 = bt_ref[1:2, pl.ds(c0, _B)]
        cx1 = bt_ref[2:3, pl.ds(c0, _B)]
        cy1 = bt_ref[3:4, pl.ds(c0, _B)]
        carea = (cx1 - cx0) * (cy1 - cy0)
        w = jnp.maximum(jnp.minimum(rx1, cx1) - jnp.maximum(rx0, cx0), 0.0)
        h = jnp.maximum(jnp.minimum(ry1, cy1) - jnp.maximum(ry0, cy0), 0.0)
        inter = w * h
        return 3.0 * inter > (rarea + carea)

    def block_body(bi, carry):
        r0 = bi * _B
        tile_ref[...] = jnp.where(
            iou_bin(r0, r0) & tri, 1.0, 0.0
        ).astype(jnp.bfloat16)
        active = 1.0 - supp_ref[:, pl.ds(r0, _B)]

        def w_cond(c):
            return c[1]

        def w_body(c):
            k, _ = c
            t = jnp.dot(
                k.astype(jnp.bfloat16),
                tile_ref[...],
                preferred_element_type=jnp.float32,
            )
            k_new = active * jnp.where(t < 0.5, 1.0, 0.0)
            return k_new, jnp.any(k_new != k)

        k, _ = jax.lax.while_loop(w_cond, w_body, (active, jnp.bool_(True)))
        supp_ref[:, pl.ds(r0, _B)] = 1.0 - k
        kb = k.astype(jnp.bfloat16)

        def col_body(j, carry2):
            c0 = j * _B
            tile = jnp.where(iou_bin(r0, c0), 1.0, 0.0).astype(jnp.bfloat16)
            t = jnp.dot(kb, tile, preferred_element_type=jnp.float32)
            supp_ref[:, pl.ds(c0, _B)] = jnp.maximum(
                supp_ref[:, pl.ds(c0, _B)],
                jnp.where(t > 0.5, 1.0, 0.0),
            )
            return carry2

        jax.lax.fori_loop(bi + 1, _NB, col_body, carry)
        return carry

    jax.lax.fori_loop(0, _NB, block_body, 0)
    keep_ref[...] = 1.0 - supp_ref[...]


def _mask_kernel(raw_ref, keep_ref, out_ref):
    out_ref[...] = raw_ref[...] * keep_ref[...]


def kernel(contours, scores, boxes, uncertainties):
    scores_row = scores.reshape(1, _N)
    unc_t = uncertainties.T
    w_row = pl.pallas_call(
        _weight_kernel,
        out_shape=jax.ShapeDtypeStruct((1, _N), jnp.float32),
    )(scores_row, unc_t)
    wflat = w_row.reshape(_N)
    order = jnp.argsort(-wflat)
    boxes_sorted = boxes[order]
    boxes_s = jnp.zeros((_NPAD, 4), jnp.float32).at[:_N].set(boxes_sorted)
    boxes_t = jnp.zeros((8, _NPAD), jnp.float32).at[:4, :_N].set(boxes_sorted.T)
    keep_row = pl.pallas_call(
        _nms_kernel,
        out_shape=jax.ShapeDtypeStruct((1, _NPAD), jnp.float32),
        scratch_shapes=[
            pltpu.VMEM((_B, _B), jnp.bfloat16),
            pltpu.VMEM((1, _NPAD), jnp.float32),
        ],
    )(boxes_s, boxes_t)
    keep = jnp.zeros(_N, jnp.float32).at[order].set(keep_row[0, :_N])
    raw = jnp.concatenate(
        [
            boxes,
            scores[:, None],
            uncertainties,
            wflat[:, None],
            contours.reshape(_N, -1),
        ],
        axis=1,
    )
    out = pl.pallas_call(
        _mask_kernel,
        out_shape=jax.ShapeDtypeStruct((_N, 74), jnp.float32),
    )(raw, keep[:, None])
    return out


# speedup vs baseline: 98.4134x; 1.1667x over previous
import jax
import jax.numpy as jnp
from jax.experimental import pallas as pl
from jax.experimental.pallas import tpu as pltpu

_N = 5000
_B = 512
_NPAD = 5120
_NB = _NPAD // _B


def _weight_kernel(scores_ref, unc_ref, w_ref):
    m = jnp.mean(unc_ref[...], axis=0, keepdims=True)
    w_ref[...] = scores_ref[...] * (1.0 - jax.nn.sigmoid(m))


def _nms_kernel(boxes_ref, bt_ref, keep_ref, tile_ref, supp_ref):
    tri = (
        jax.lax.broadcasted_iota(jnp.int32, (_B, _B), 1)
        > jax.lax.broadcasted_iota(jnp.int32, (_B, _B), 0)
    )
    supp_ref[...] = jnp.zeros((1, _NPAD), jnp.float32)

    def iou_bin(r0, c0):
        rows = boxes_ref[pl.ds(r0, _B), :]
        rx0 = rows[:, 0:1]
        ry0 = rows[:, 1:2]
        rx1 = rows[:, 2:3]
        ry1 = rows[:, 3:4]
        rarea = (rx1 - rx0) * (ry1 - ry0)
        cx0 = bt_ref[0:1, pl.ds(c0, _B)]
        cy0 = bt_ref[1:2, pl.ds(c0, _B)]
        cx1 = bt_ref[2:3, pl.ds(c0, _B)]
        cy1 = bt_ref[3:4, pl.ds(c0, _B)]
        carea = (cx1 - cx0) * (cy1 - cy0)
        w = jnp.maximum(jnp.minimum(rx1, cx1) - jnp.maximum(rx0, cx0), 0.0)
        h = jnp.maximum(jnp.minimum(ry1, cy1) - jnp.maximum(ry0, cy0), 0.0)
        inter = w * h
        return 3.0 * inter > (rarea + carea)

    def block_body(bi, carry):
        r0 = bi * _B
        tile_ref[...] = jnp.where(
            iou_bin(r0, r0) & tri, 1.0, 0.0
        ).astype(jnp.bfloat16)
        active = 1.0 - supp_ref[:, pl.ds(r0, _B)]

        def w_cond(c):
            return c[1]

        def w_body(c):
            k, _ = c
            t = jnp.dot(
                k.astype(jnp.bfloat16),
                tile_ref[...],
                preferred_element_type=jnp.float32,
            )
            k_new = active * jnp.where(t < 0.5, 1.0, 0.0)
            return k_new, jnp.any(k_new != k)

        k, _ = jax.lax.while_loop(w_cond, w_body, (active, jnp.bool_(True)))
        supp_ref[:, pl.ds(r0, _B)] = 1.0 - k
        kb = k.astype(jnp.bfloat16)

        def col_body(j, carry2):
            c0 = j * _B
            tile = jnp.where(iou_bin(r0, c0), 1.0, 0.0).astype(jnp.bfloat16)
            t = jnp.dot(kb, tile, preferred_element_type=jnp.float32)
            supp_ref[:, pl.ds(c0, _B)] = jnp.maximum(
                supp_ref[:, pl.ds(c0, _B)],
                jnp.where(t > 0.5, 1.0, 0.0),
            )
            return carry2

        jax.lax.fori_loop(bi + 1, _NB, col_body, carry)
        return carry

    jax.lax.fori_loop(0, _NB, block_body, 0)
    keep_ref[...] = 1.0 - supp_ref[...]


def _assemble_kernel(boxes_ref, scores_ref, unc_ref, w_ref, cont_ref, keep_ref, out_ref):
    k = keep_ref[...]
    out_ref[:, 0:4] = boxes_ref[...] * k
    out_ref[:, 4:5] = scores_ref[...] * k
    out_ref[:, 5:9] = unc_ref[...] * k
    out_ref[:, 9:10] = w_ref[...] * k
    out_ref[:, 10:74] = cont_ref[...] * k


def kernel(contours, scores, boxes, uncertainties):
    scores_row = scores.reshape(1, _N)
    unc_t = uncertainties.T
    w_row = pl.pallas_call(
        _weight_kernel,
        out_shape=jax.ShapeDtypeStruct((1, _N), jnp.float32),
    )(scores_row, unc_t)
    wflat = w_row.reshape(_N)
    order = jnp.argsort(-wflat)
    boxes_sorted = boxes[order]
    boxes_s = jnp.zeros((_NPAD, 4), jnp.float32).at[:_N].set(boxes_sorted)
    boxes_t = jnp.zeros((8, _NPAD), jnp.float32).at[:4, :_N].set(boxes_sorted.T)
    keep_row = pl.pallas_call(
        _nms_kernel,
        out_shape=jax.ShapeDtypeStruct((1, _NPAD), jnp.float32),
        scratch_shapes=[
            pltpu.VMEM((_B, _B), jnp.bfloat16),
            pltpu.VMEM((1, _NPAD), jnp.float32),
        ],
    )(boxes_s, boxes_t)
    _, keep = jax.lax.sort(
        (order.astype(jnp.int32), keep_row[0, :_N]), num_keys=1
    )
    out = pl.pallas_call(
        _assemble_kernel,
        out_shape=jax.ShapeDtypeStruct((_N, 74), jnp.float32),
    )(
        boxes,
        scores[:, None],
        uncertainties,
        wflat[:, None],
        contours.reshape(_N, -1),
        keep[:, None],
    )
    return out


# speedup vs baseline: 103.6227x; 1.0529x over previous
import jax
import jax.numpy as jnp
from jax.experimental import pallas as pl
from jax.experimental.pallas import tpu as pltpu

_N = 5000
_B = 512
_NPAD = 5120
_NB = _NPAD // _B


def _weight_kernel(scores_ref, unc_ref, w_ref):
    m = jnp.mean(unc_ref[...], axis=0, keepdims=True)
    w_ref[...] = scores_ref[...] * (1.0 - jax.nn.sigmoid(m))


def _nms_kernel(
    xs0_ref, ys0_ref, xs1_ref, ys1_ref, a3_ref, bt_ref, keep_ref, tile_ref, supp_ref
):
    tri = (
        jax.lax.broadcasted_iota(jnp.int32, (_B, _B), 1)
        > jax.lax.broadcasted_iota(jnp.int32, (_B, _B), 0)
    )
    supp_ref[...] = jnp.zeros((1, _NPAD), jnp.float32)

    def block_body(bi, carry):
        r0 = bi * _B
        rx0 = xs0_ref[pl.ds(r0, _B), :]
        ry0 = ys0_ref[pl.ds(r0, _B), :]
        rx1 = xs1_ref[pl.ds(r0, _B), :]
        ry1 = ys1_ref[pl.ds(r0, _B), :]
        ra3 = a3_ref[pl.ds(r0, _B), :]

        def iou_bin(c0):
            cx0 = bt_ref[0:1, pl.ds(c0, _B)]
            cy0 = bt_ref[1:2, pl.ds(c0, _B)]
            cx1 = bt_ref[2:3, pl.ds(c0, _B)]
            cy1 = bt_ref[3:4, pl.ds(c0, _B)]
            ca3 = bt_ref[4:5, pl.ds(c0, _B)]
            w = jnp.maximum(jnp.minimum(rx1, cx1) - jnp.maximum(rx0, cx0), 0.0)
            h = jnp.maximum(jnp.minimum(ry1, cy1) - jnp.maximum(ry0, cy0), 0.0)
            return w * h > (ra3 + ca3)

        tile_ref[...] = jnp.where(
            iou_bin(r0) & tri, 1.0, 0.0
        ).astype(jnp.bfloat16)
        active = 1.0 - supp_ref[:, pl.ds(r0, _B)]

        def w_cond(c):
            return c[1]

        def w_body(c):
            k, _ = c
            t = jnp.dot(
                k.astype(jnp.bfloat16),
                tile_ref[...],
                preferred_element_type=jnp.float32,
            )
            k_new = active * jnp.where(t < 0.5, 1.0, 0.0)
            return k_new, jnp.any(k_new != k)

        k, _ = jax.lax.while_loop(w_cond, w_body, (active, jnp.bool_(True)))
        supp_ref[:, pl.ds(r0, _B)] = 1.0 - k
        kb = k.astype(jnp.bfloat16)

        def col_body(j, carry2):
            c0 = j * _B
            tile = jnp.where(iou_bin(c0), 1.0, 0.0).astype(jnp.bfloat16)
            t = jnp.dot(kb, tile, preferred_element_type=jnp.float32)
            supp_ref[:, pl.ds(c0, _B)] = jnp.maximum(
                supp_ref[:, pl.ds(c0, _B)],
                jnp.where(t > 0.5, 1.0, 0.0),
            )
            return carry2

        jax.lax.fori_loop(bi + 1, _NB, col_body, carry)
        return carry

    jax.lax.fori_loop(0, _NB, block_body, 0)
    keep_ref[...] = 1.0 - supp_ref[...]


def _assemble_kernel(boxes_ref, scores_ref, unc_ref, w_ref, cont_ref, keep_ref, out_ref):
    k = keep_ref[...]
    out_ref[:, 0:4] = boxes_ref[...] * k
    out_ref[:, 4:5] = scores_ref[...] * k
    out_ref[:, 5:9] = unc_ref[...] * k
    out_ref[:, 9:10] = w_ref[...] * k
    out_ref[:, 10:74] = cont_ref[...] * k


def kernel(contours, scores, boxes, uncertainties):
    scores_row = scores.reshape(1, _N)
    unc_t = uncertainties.T
    w_row = pl.pallas_call(
        _weight_kernel,
        out_shape=jax.ShapeDtypeStruct((1, _N), jnp.float32),
    )(scores_row, unc_t)
    wflat = w_row.reshape(_N)
    order = jnp.argsort(-wflat)
    boxes_sorted = boxes[order]
    area3 = (
        (boxes_sorted[:, 2] - boxes_sorted[:, 0])
        * (boxes_sorted[:, 3] - boxes_sorted[:, 1])
        / 3.0
    )
    cols_sorted = jnp.concatenate([boxes_sorted, area3[:, None]], axis=1)
    cols_pad = jnp.zeros((_NPAD, 5), jnp.float32).at[:_N].set(cols_sorted)
    boxes_t = jnp.zeros((8, _NPAD), jnp.float32).at[:5, :_N].set(cols_sorted.T)
    keep_row = pl.pallas_call(
        _nms_kernel,
        out_shape=jax.ShapeDtypeStruct((1, _NPAD), jnp.float32),
        scratch_shapes=[
            pltpu.VMEM((_B, _B), jnp.bfloat16),
            pltpu.VMEM((1, _NPAD), jnp.float32),
        ],
    )(
        cols_pad[:, 0:1],
        cols_pad[:, 1:2],
        cols_pad[:, 2:3],
        cols_pad[:, 3:4],
        cols_pad[:, 4:5],
        boxes_t,
    )
    _, keep = jax.lax.sort(
        (order.astype(jnp.int32), keep_row[0, :_N]), num_keys=1
    )
    out = pl.pallas_call(
        _assemble_kernel,
        out_shape=jax.ShapeDtypeStruct((_N, 74), jnp.float32),
    )(
        boxes,
        scores[:, None],
        uncertainties,
        wflat[:, None],
        contours.reshape(_N, -1),
        keep[:, None],
    )
    return out
